# Initial kernel scaffold; baseline (speedup 1.0000x reference)
import jax
import jax.numpy as jnp
from jax.experimental import pallas as pl


def kernel(Y, D):
    raise NotImplementedError("write your pallas kernel here")


# speedup vs baseline: 5.4685x; 5.4685x over previous
import jax
import jax.numpy as jnp
from jax import lax
from jax.experimental import pallas as pl

N = 512
M = 128
KMAX = 8
DIAG_EPS = 1e-5
TB = 256


def _trunc_bf16(v):
    t = lax.bitcast_convert_type(v, jnp.int32)
    r = (t + 0x8000 + ((t >> 16) & 1)) & jnp.int32(-65536)
    return lax.bitcast_convert_type(r, jnp.float32)


def _group_sum(ps):
    if len(ps) == 1:
        return ps[0]
    m = jnp.abs(ps[0])
    for p in ps[1:]:
        m = jnp.maximum(m, jnp.abs(p))
    emax = (lax.bitcast_convert_type(m, jnp.int32) >> 23) & 0xFF
    total = jnp.zeros_like(emax)
    for p in ps:
        t = lax.bitcast_convert_type(p, jnp.int32)
        ej = (t >> 23) & 0xFF
        mant = jnp.where(ej == 0, 0, (t & 0x7FFFFF) | 0x800000)
        d = emax - ej
        up = jnp.clip(4 - d, 0, 4)
        dn = jnp.clip(d - 4, 0, 31)
        mag = (mant << up) >> dn
        total = total + jnp.where(t < 0, -mag, mag)
    qexp = jnp.clip(emax - 27, 1, 254)
    q = lax.bitcast_convert_type(qexp << 23, jnp.float32)
    return total.astype(jnp.float32) * q


def _gram_kernel(d_ref, g_ref):
    d = d_ref[...]
    g = lax.dot_general(d, d, (((1,), (1,)), ((), ())),
                        preferred_element_type=jnp.float32)
    row = lax.broadcasted_iota(jnp.int32, (N, N), 0)
    col = lax.broadcasted_iota(jnp.int32, (N, N), 1)
    g_ref[...] = g + jnp.where(row == col, DIAG_EPS, 0.0).astype(jnp.float32)


def _omp_kernel(y_ref, d_ref, g_ref, xt_ref, ypt_ref):
    d = d_ref[...]
    g = g_ref[...]
    y = y_ref[...]
    hbar = lax.dot_general(d, y, (((1,), (1,)), ((), ())),
                           preferred_element_type=jnp.float32)
    grow_dg = lax.broadcasted_iota(jnp.int32, (N, N), 0)
    gcol_dg = lax.broadcasted_iota(jnp.int32, (N, N), 1)
    gdiag = jnp.sum(jnp.where(grow_dg == gcol_dg, g, 0.0),
                    axis=1, keepdims=True)
    iota = lax.broadcasted_iota(jnp.int32, (N, TB), 0)

    selmask = jnp.zeros((N, TB), jnp.bool_)
    h = hbar
    idxs = []
    grows = []
    grows_t = []
    hsel = []
    xs = []
    L = {}

    for k in range(1, KMAX + 1):
        a = jnp.where(selmask, 0.0, jnp.abs(h))
        mx = jnp.max(a, axis=0, keepdims=True)
        cand = jnp.where(a == mx, iota, N)
        index = jnp.min(cand, axis=0, keepdims=True)
        onehot = iota == index
        onef = onehot.astype(jnp.float32)
        selmask = jnp.logical_or(selmask, onehot)

        diag_g = jnp.sum(gdiag * onef, axis=0, keepdims=True)
        hs = jnp.sum(hbar * onef, axis=0, keepdims=True)
        idxs.append(index)
        hsel.append(hs)

        if k == 1:
            L[(0, 0)] = jnp.ones_like(hs)
        else:
            b = [jnp.sum(grows[j] * onef, axis=0, keepdims=True)
                 for j in range(k - 1)]
            w = []
            for i in range(k - 1):
                acc = b[i]
                for j in range(i):
                    acc = acc - L[(i, j)] * w[j]
                w.append(acc / L[(i, i)])
            ssum = w[0] * w[0]
            for j in range(1, k - 1):
                ssum = ssum + w[j] * w[j]
            corner = jnp.sqrt(jnp.maximum(diag_g - ssum, DIAG_EPS))
            for j in range(k - 1):
                L[(k - 1, j)] = w[j]
            L[(k - 1, k - 1)] = corner

        ysol = []
        for i in range(k):
            acc = hsel[i]
            for j in range(i):
                acc = acc - L[(i, j)] * ysol[j]
            ysol.append(acc / L[(i, i)])
        xs = [None] * k
        for i in reversed(range(k)):
            acc = ysol[i]
            for j in range(i + 1, k):
                acc = acc - L[(j, i)] * xs[j]
            xs[i] = acc / L[(i, i)]

        if k < KMAX:
            growk = lax.dot_general(g, onef, (((0,), (0,)), ((), ())),
                                    preferred_element_type=jnp.float32,
                                    precision=lax.Precision.HIGHEST)
            grows.append(growk)
            grows_t.append(_trunc_bf16(growk))
            prods = [_trunc_bf16(xs[j]) * grows_t[j] for j in range(k)]
            beta = _group_sum(prods[0:4])
            for g0 in range(4, k, 4):
                beta = beta + _group_sum(prods[g0:g0 + 4])
            h = hbar - beta

    xt = jnp.zeros((N, TB), jnp.float32)
    for j in range(KMAX):
        xt = xt + xs[j] * (iota == idxs[j]).astype(jnp.float32)
    xt_ref[...] = xt
    ypt_ref[...] = lax.dot_general(d, xt, (((0,), (0,)), ((), ())),
                                   preferred_element_type=jnp.float32)


def kernel(Y, D):
    B = Y.shape[0]
    G = pl.pallas_call(
        _gram_kernel,
        out_shape=jax.ShapeDtypeStruct((N, N), jnp.float32),
    )(D)
    XT, YPT = pl.pallas_call(
        _omp_kernel,
        grid=(B // TB,),
        in_specs=[
            pl.BlockSpec((TB, M), lambda i: (i, 0)),
            pl.BlockSpec((N, M), lambda i: (0, 0)),
            pl.BlockSpec((N, N), lambda i: (0, 0)),
        ],
        out_specs=[
            pl.BlockSpec((N, TB), lambda i: (0, i)),
            pl.BlockSpec((M, TB), lambda i: (0, i)),
        ],
        out_shape=[
            jax.ShapeDtypeStruct((N, B), jnp.float32),
            jax.ShapeDtypeStruct((M, B), jnp.float32),
        ],
    )(Y, D, G)
    return YPT.T, XT.T
